# Initial kernel scaffold; baseline (speedup 1.0000x reference)
import jax
import jax.numpy as jnp
from jax.experimental import pallas as pl


def kernel(nodes, edge_attr, params, edge_index):
    raise NotImplementedError("write your pallas kernel here")


# speedup vs baseline: 2.6121x; 2.6121x over previous
import functools

import jax
import jax.numpy as jnp
from jax import lax
from jax.experimental import pallas as pl
from jax.experimental.pallas import tpu as pltpu
from jax.experimental.pallas import tpu_sc as plsc

_N = 10000
_E = 320000
_DH = 64
_NW = 32
_RPW = _E // _NW
_CH = 80
_NCH = _RPW // _CH
_NB = 4
_SNB = 2
_NPAD = 10240
_ZR = _NPAD // 16
_WLAST = _N - 15 * _ZR

_NBLK = 1000
_EBLK = 2000

_f32 = jnp.float32


def _ln(x, scale, bias):
    mu = jnp.mean(x, axis=-1, keepdims=True)
    var = jnp.mean((x - mu) * (x - mu), axis=-1, keepdims=True)
    return (x - mu) / jnp.sqrt(var + 1e-6) * scale + bias


_sc_mesh = plsc.VectorSubcoreMesh(core_axis_name="c", subcore_axis_name="s")


@functools.partial(
    pl.kernel,
    mesh=_sc_mesh,
    out_type=[
        jax.ShapeDtypeStruct((_E, 2 * _DH), _f32),
        jax.ShapeDtypeStruct((_E, 2 * _DH), _f32),
    ],
    scratch_types=(
        [pltpu.VMEM((_RPW,), jnp.int32)] * 2
        + [pltpu.VMEM((_CH, 2 * _DH), _f32)] * (2 * _NB)
        + [pltpu.SemaphoreType.DMA] * (4 * _NB)
    ),
)
def _sc_gather(tab_hbm, sidx_hbm, ridx_hbm, gs_hbm, gr_hbm,
               sidx_v, ridx_v, *rest):
    bufs = rest[:_NB]
    bufr = rest[_NB:2 * _NB]
    gsem_s = rest[2 * _NB:3 * _NB]
    gsem_r = rest[3 * _NB:4 * _NB]
    wsem_s = rest[4 * _NB:5 * _NB]
    wsem_r = rest[5 * _NB:6 * _NB]

    c = lax.axis_index("c")
    s = lax.axis_index("s")
    w = c * 16 + s
    base = w * _RPW

    pltpu.sync_copy(sidx_hbm.at[w], sidx_v)
    pltpu.sync_copy(ridx_hbm.at[w], ridx_v)

    def fire_gather(j, b):
        ds = pl.ds(j * _CH, _CH)
        pltpu.async_copy(tab_hbm.at[sidx_v.at[ds]], bufs[b], gsem_s[b])
        pltpu.async_copy(tab_hbm.at[ridx_v.at[ds]], bufr[b], gsem_r[b])

    def wait_gather(j, b):
        ds = pl.ds(j * _CH, _CH)
        pltpu.make_async_copy(tab_hbm.at[sidx_v.at[ds]], bufs[b], gsem_s[b]).wait()
        pltpu.make_async_copy(tab_hbm.at[ridx_v.at[ds]], bufr[b], gsem_r[b]).wait()

    def fire_write(j, b):
        dst = pl.ds(base + j * _CH, _CH)
        pltpu.async_copy(bufs[b], gs_hbm.at[dst], wsem_s[b])
        pltpu.async_copy(bufr[b], gr_hbm.at[dst], wsem_r[b])

    def wait_write(j, b):
        dst = pl.ds(base + j * _CH, _CH)
        pltpu.make_async_copy(bufs[b], gs_hbm.at[dst], wsem_s[b]).wait()
        pltpu.make_async_copy(bufr[b], gr_hbm.at[dst], wsem_r[b]).wait()

    def outer(j, carry):
        fire_gather(j, 0)
        wait_gather(j, 0)
        fire_write(j, 0)
        wait_write(j, 0)
        return carry

    lax.fori_loop(0, _NCH, outer, 0)


@functools.partial(
    pl.kernel,
    mesh=_sc_mesh,
    out_type=jax.ShapeDtypeStruct((2 * _N, 2 * _DH), _f32),
    scratch_types=(
        [pltpu.VMEM((_NCH, _CH), jnp.int32),
         pltpu.VMEM((_CH, 2 * _DH), _f32)]
        + [pltpu.VMEM((_CH, 2 * _DH), _f32)] * _SNB
        + [pltpu.VMEM_SHARED((_NPAD, 2 * _DH), _f32)]
        + [pltpu.SemaphoreType.DMA] * _SNB
    ),
)
def _sc_scatter(e_hbm, ridx_hbm, out_hbm, ridx_v, zbuf, *rest):
    bufs = rest[:_SNB]
    shared = rest[_SNB]
    lsem = rest[_SNB + 1:]

    c = lax.axis_index("c")
    s = lax.axis_index("s")
    w = c * 16 + s
    base = w * _RPW

    pltpu.sync_copy(ridx_hbm.at[w], ridx_v)

    def zrow(i, carry):
        for q in range(8):
            zbuf[i, pl.ds(q * 16, 16)] = jnp.zeros((16,), _f32)
        return carry

    lax.fori_loop(0, _CH, zrow, 0)
    for q in range(_ZR // _CH):
        pltpu.sync_copy(zbuf, shared.at[pl.ds(s * _ZR + q * _CH, _CH)])
    plsc.subcore_barrier()

    def fire_load(j, b):
        pltpu.async_copy(e_hbm.at[pl.ds(base + j * _CH, _CH)], bufs[b], lsem[b])

    def wait_load(j, b):
        pltpu.make_async_copy(
            e_hbm.at[pl.ds(base + j * _CH, _CH)], bufs[b], lsem[b]).wait()

    def outer(j, carry):
        fire_load(j, 0)
        wait_load(j, 0)
        pltpu.sync_copy(bufs[0], shared.at[ridx_v.at[j]], add=True)
        return carry

    lax.fori_loop(0, _NCH, outer, 0)

    plsc.subcore_barrier()

    @pl.when(s < 15)
    def _():
        pltpu.sync_copy(shared.at[pl.ds(s * _ZR, _ZR)],
                        out_hbm.at[pl.ds(c * _N + s * _ZR, _ZR)])

    @pl.when(s == 15)
    def _():
        pltpu.sync_copy(shared.at[pl.ds(15 * _ZR, _WLAST)],
                        out_hbm.at[pl.ds(c * _N + 15 * _ZR, _WLAST)])


def _dot(a, b):
    return jnp.dot(a, b, preferred_element_type=_f32)


def _full(shape):
    return pl.BlockSpec(shape, lambda *a: tuple(0 for _ in shape))


def _rows(shape):
    return pl.BlockSpec(shape, lambda i: (i,) + tuple(0 for _ in shape[1:]))


def _encoder_call(nodes, We, be, Wproj):
    def body(n_ref, we_ref, be_ref, wp_ref, x_ref, t_ref):
        x = _dot(n_ref[...], we_ref[...]) + be_ref[...]
        x_ref[...] = x
        t_ref[...] = _dot(x, wp_ref[...])

    return pl.pallas_call(
        body,
        grid=(_N // _NBLK,),
        in_specs=[_rows((_NBLK, 128)), _full((128, _DH)), _full((1, _DH)),
                  _full((_DH, 2 * _DH))],
        out_specs=[_rows((_NBLK, _DH)), _rows((_NBLK, 2 * _DH))],
        out_shape=[jax.ShapeDtypeStruct((_N, _DH), _f32),
                   jax.ShapeDtypeStruct((_N, 2 * _DH), _f32)],
    )(nodes, We, be.reshape(1, _DH), Wproj)


def _edge_call(e, gs, gr, wb, ln, d_store):
    (W1e, b1), (W2, b2), (W3, b3) = wb

    def body(*refs):
        if ln is None:
            (e_ref, gs_ref, gr_ref, w1_ref, b1_ref, w2_ref, b2_ref,
             w3_ref, b3_ref, out_ref) = refs
            ein = e_ref[...]
        else:
            (e_ref, gs_ref, gr_ref, lns_ref, lnb_ref, w1_ref, b1_ref,
             w2_ref, b2_ref, w3_ref, b3_ref, out_ref) = refs
            ein = _ln(e_ref[:, :_DH], lns_ref[...], lnb_ref[...])
        g = gs_ref[:, :_DH] + gr_ref[:, _DH:]
        h = _dot(ein, w1_ref[...]) + g + b1_ref[...]
        h = jax.nn.gelu(h)
        h = jax.nn.gelu(_dot(h, w2_ref[...]) + b2_ref[...])
        h = _dot(h, w3_ref[...]) + b3_ref[...]
        out_ref[...] = jnp.concatenate([h, jnp.zeros_like(h)], axis=-1)

    d_in = W1e.shape[0]
    ins = [e, gs, gr]
    in_specs = [_rows((_EBLK, d_store)),
                _rows((_EBLK, 2 * _DH)), _rows((_EBLK, 2 * _DH))]
    if ln is not None:
        ins += [ln[0].reshape(1, _DH), ln[1].reshape(1, _DH)]
        in_specs += [_full((1, _DH))] * 2
    ins += [W1e, b1.reshape(1, _DH), W2, b2.reshape(1, _DH),
            W3, b3.reshape(1, _DH)]
    in_specs += [_full((d_in, _DH)), _full((1, _DH)), _full((_DH, _DH)),
                 _full((1, _DH)), _full((_DH, _DH)), _full((1, _DH))]
    return pl.pallas_call(
        body,
        grid=(_E // _EBLK,),
        in_specs=in_specs,
        out_specs=_rows((_EBLK, 2 * _DH)),
        out_shape=jax.ShapeDtypeStruct((_E, 2 * _DH), _f32),
    )(*ins)


def _node_call(x, parts, wb, lnn, proj):
    (W1, b1), (W2, b2), (W3, b3) = wb
    W1x, W1r = W1[:_DH], W1[_DH:]

    def body(*refs):
        if proj is None:
            (x_ref, p0_ref, p1_ref, w1x_ref, w1r_ref, b1_ref, w2_ref, b2_ref,
             w3_ref, b3_ref, lns_ref, lnb_ref, xo_ref) = refs
        else:
            (x_ref, p0_ref, p1_ref, w1x_ref, w1r_ref, b1_ref, w2_ref, b2_ref,
             w3_ref, b3_ref, lns_ref, lnb_ref, wp_ref,
             xo_ref, t_ref) = refs
        recv = p0_ref[:, :_DH] + p1_ref[:, :_DH]
        h = jax.nn.gelu(_dot(x_ref[...], w1x_ref[...])
                        + _dot(recv, w1r_ref[...]) + b1_ref[...])
        h = jax.nn.gelu(_dot(h, w2_ref[...]) + b2_ref[...])
        xn = _dot(h, w3_ref[...]) + b3_ref[...]
        xo = _ln(xn, lns_ref[...], lnb_ref[...])
        xo_ref[...] = xo
        if proj is not None:
            t_ref[...] = _dot(xo, wp_ref[...])

    nb = _N // _NBLK
    ins = [x, parts, parts, W1x, W1r, b1.reshape(1, _DH),
           W2, b2.reshape(1, _DH), W3, b3.reshape(1, _DH),
           lnn[0].reshape(1, _DH), lnn[1].reshape(1, _DH)]
    in_specs = ([_rows((_NBLK, _DH)),
                 pl.BlockSpec((_NBLK, 2 * _DH), lambda i: (i, 0)),
                 pl.BlockSpec((_NBLK, 2 * _DH), lambda i: (i + nb, 0))]
                + [_full((_DH, _DH)), _full((_DH, _DH)), _full((1, _DH)),
                   _full((_DH, _DH)), _full((1, _DH)), _full((_DH, _DH)),
                   _full((1, _DH)), _full((1, _DH)), _full((1, _DH))])
    o = jax.ShapeDtypeStruct((_N, _DH), _f32)
    if proj is None:
        out_specs, out_shape = _rows((_NBLK, _DH)), o
    else:
        ins += [proj]
        in_specs += [_full((_DH, 2 * _DH))]
        out_specs = [_rows((_NBLK, _DH)), _rows((_NBLK, 2 * _DH))]
        out_shape = [o, jax.ShapeDtypeStruct((_N, 2 * _DH), _f32)]
    return pl.pallas_call(
        body,
        grid=(_N // _NBLK,),
        in_specs=in_specs,
        out_specs=out_specs,
        out_shape=out_shape,
    )(*ins)


def _readout_call(x, ps):
    (W1, b1), (W2, b2), (W3, b3), (W4, b4) = ps

    def body(x_ref, w1_ref, b1_ref, w2_ref, b2_ref, w3_ref, b3_ref,
             w4t_ref, b4_ref, out_ref):
        agg = jnp.mean(x_ref[...], axis=0, keepdims=True)
        h = jax.nn.gelu(_dot(agg, w1_ref[...]) + b1_ref[...])
        h = jax.nn.gelu(_dot(h, w2_ref[...]) + b2_ref[...])
        h = jax.nn.gelu(_dot(h, w3_ref[...]) + b3_ref[...])
        y = jnp.sum(h * w4t_ref[...]) + b4_ref[0, 0]
        out_ref[...] = jnp.full((1, 128), y, _f32)

    out = pl.pallas_call(
        body,
        in_specs=[_full((_N, _DH)), _full((_DH, 4 * _DH)), _full((1, 4 * _DH)),
                  _full((4 * _DH, 2 * _DH)), _full((1, 2 * _DH)),
                  _full((2 * _DH, 2 * _DH)), _full((1, 2 * _DH)),
                  _full((1, 2 * _DH)), _full((1, 1))],
        out_specs=_full((1, 128)),
        out_shape=jax.ShapeDtypeStruct((1, 128), _f32),
    )(x, W1, b1.reshape(1, -1), W2, b2.reshape(1, -1), W3, b3.reshape(1, -1),
      W4.reshape(1, -1), b4.reshape(1, 1))
    return out[0, :1]


def kernel(nodes, edge_attr, params, edge_index):
    senders = edge_index[0].reshape(_NW, _RPW)
    receivers = edge_index[1].reshape(_NW, _RPW)
    recv_sc = edge_index[1].reshape(_NW, _NCH, _CH)
    steps = params['steps']

    def _proj_w(W1, d_e):
        return jnp.concatenate([W1[d_e:d_e + _DH], W1[d_e + _DH:]], axis=1)

    w1_0 = steps[0]['edge'][0][0]
    x, tab = _encoder_call(nodes, params['enc'][0], params['enc'][1],
                           _proj_w(w1_0, 16))

    e = edge_attr
    for s in range(len(steps)):
        stp = steps[s]
        d_e = 16 if s == 0 else _DH
        d_store = 16 if s == 0 else 2 * _DH
        (W1, b1), l2, l3 = stp['edge']
        wb = ((W1[:d_e], b1), l2, l3)
        ln = None if s == 0 else (steps[s - 1]['ln_e'][0], steps[s - 1]['ln_e'][1])

        gs, gr = _sc_gather(tab, senders, receivers)
        e = _edge_call(e, gs, gr, wb, ln, d_store)
        parts = _sc_scatter(e, recv_sc)
        if s + 1 < len(steps):
            proj = _proj_w(steps[s + 1]['edge'][0][0], _DH)
            x, tab = _node_call(x, parts, stp['node'], stp['ln_n'], proj)
        else:
            x = _node_call(x, parts, stp['node'], stp['ln_n'], None)

    return _readout_call(x, params['readout'])


# speedup vs baseline: 3.1288x; 1.1978x over previous
import functools

import jax
import jax.numpy as jnp
from jax import lax
from jax.experimental import pallas as pl
from jax.experimental.pallas import tpu as pltpu
from jax.experimental.pallas import tpu_sc as plsc

_N = 10000
_E = 320000
_DH = 64
_NW = 32
_RPW = _E // _NW
_CH = 80
_NCH = _RPW // _CH
_NB = 5
_SNB = 2
_NPAD = 10240
_ZR = _NPAD // 16
_WLAST = _N - 15 * _ZR

_NBLK = 1000
_EBLK = 2000

_f32 = jnp.float32


def _ln(x, scale, bias):
    mu = jnp.mean(x, axis=-1, keepdims=True)
    var = jnp.mean((x - mu) * (x - mu), axis=-1, keepdims=True)
    return (x - mu) / jnp.sqrt(var + 1e-6) * scale + bias


_sc_mesh = plsc.VectorSubcoreMesh(core_axis_name="c", subcore_axis_name="s")


@functools.partial(
    pl.kernel,
    mesh=_sc_mesh,
    out_type=[
        jax.ShapeDtypeStruct((_E, 2 * _DH), _f32),
        jax.ShapeDtypeStruct((_E, 2 * _DH), _f32),
    ],
    scratch_types=(
        [pltpu.VMEM((_RPW,), jnp.int32)] * 2
        + [pltpu.VMEM((_CH, 2 * _DH), _f32)] * (2 * _NB)
        + [pltpu.SemaphoreType.DMA] * (4 * _NB)
    ),
)
def _sc_gather(tab_hbm, sidx_hbm, ridx_hbm, gs_hbm, gr_hbm,
               sidx_v, ridx_v, *rest):
    bufs = rest[:_NB]
    bufr = rest[_NB:2 * _NB]
    gsem_s = rest[2 * _NB:3 * _NB]
    gsem_r = rest[3 * _NB:4 * _NB]
    wsem_s = rest[4 * _NB:5 * _NB]
    wsem_r = rest[5 * _NB:6 * _NB]

    c = lax.axis_index("c")
    s = lax.axis_index("s")
    w = c * 16 + s
    base = w * _RPW

    pltpu.sync_copy(sidx_hbm.at[w], sidx_v)
    pltpu.sync_copy(ridx_hbm.at[w], ridx_v)

    def fire_gather(j, b):
        ds = pl.ds(j * _CH, _CH)
        pltpu.async_copy(tab_hbm.at[sidx_v.at[ds]], bufs[b], gsem_s[b])
        pltpu.async_copy(tab_hbm.at[ridx_v.at[ds]], bufr[b], gsem_r[b])

    def wait_gather(j, b):
        ds = pl.ds(j * _CH, _CH)
        pltpu.make_async_copy(tab_hbm.at[sidx_v.at[ds]], bufs[b], gsem_s[b]).wait()
        pltpu.make_async_copy(tab_hbm.at[ridx_v.at[ds]], bufr[b], gsem_r[b]).wait()

    def fire_write(j, b):
        dst = pl.ds(base + j * _CH, _CH)
        pltpu.async_copy(bufs[b], gs_hbm.at[dst], wsem_s[b])
        pltpu.async_copy(bufr[b], gr_hbm.at[dst], wsem_r[b])

    def wait_write(j, b):
        dst = pl.ds(base + j * _CH, _CH)
        pltpu.make_async_copy(bufs[b], gs_hbm.at[dst], wsem_s[b]).wait()
        pltpu.make_async_copy(bufr[b], gr_hbm.at[dst], wsem_r[b]).wait()

    for b in range(_NB):
        fire_gather(b, b)

    def outer(g, carry):
        for b in range(_NB):
            j = g * _NB + b
            wait_gather(j, b)
            fire_write(j, b)
        for b in range(_NB):
            j = g * _NB + b
            wait_write(j, b)

            @pl.when(j + _NB < _NCH)
            def _():
                fire_gather(j + _NB, b)
        return carry

    lax.fori_loop(0, _NCH // _NB, outer, 0)


@functools.partial(
    pl.kernel,
    mesh=_sc_mesh,
    out_type=jax.ShapeDtypeStruct((2 * _N, 2 * _DH), _f32),
    scratch_types=(
        [pltpu.VMEM((_NCH, _CH), jnp.int32),
         pltpu.VMEM((_CH, 2 * _DH), _f32)]
        + [pltpu.VMEM((_CH, 2 * _DH), _f32)] * _SNB
        + [pltpu.VMEM_SHARED((_NPAD, 2 * _DH), _f32)]
        + [pltpu.SemaphoreType.DMA] * _SNB
    ),
)
def _sc_scatter(e_hbm, ridx_hbm, out_hbm, ridx_v, zbuf, *rest):
    bufs = rest[:_SNB]
    shared = rest[_SNB]
    lsem = rest[_SNB + 1:]

    c = lax.axis_index("c")
    s = lax.axis_index("s")
    w = c * 16 + s
    base = w * _RPW

    pltpu.sync_copy(ridx_hbm.at[w], ridx_v)

    def zrow(i, carry):
        for q in range(8):
            zbuf[i, pl.ds(q * 16, 16)] = jnp.zeros((16,), _f32)
        return carry

    lax.fori_loop(0, _CH, zrow, 0)
    for q in range(_ZR // _CH):
        pltpu.sync_copy(zbuf, shared.at[pl.ds(s * _ZR + q * _CH, _CH)])
    plsc.subcore_barrier()

    def fire_load(j, b):
        pltpu.async_copy(e_hbm.at[pl.ds(base + j * _CH, _CH)], bufs[b], lsem[b])

    def wait_load(j, b):
        pltpu.make_async_copy(
            e_hbm.at[pl.ds(base + j * _CH, _CH)], bufs[b], lsem[b]).wait()

    for b in range(_SNB):
        fire_load(b, b)

    def outer(g, carry):
        for b in range(_SNB):
            j = g * _SNB + b

            @pl.when(j < _NCH)
            def _():
                wait_load(j, b)
                pltpu.sync_copy(bufs[b], shared.at[ridx_v.at[j]], add=True)

            @pl.when(j + _SNB < _NCH)
            def _():
                fire_load(j + _SNB, b)
        return carry

    lax.fori_loop(0, (_NCH + _SNB - 1) // _SNB, outer, 0)

    plsc.subcore_barrier()

    @pl.when(s < 15)
    def _():
        pltpu.sync_copy(shared.at[pl.ds(s * _ZR, _ZR)],
                        out_hbm.at[pl.ds(c * _N + s * _ZR, _ZR)])

    @pl.when(s == 15)
    def _():
        pltpu.sync_copy(shared.at[pl.ds(15 * _ZR, _WLAST)],
                        out_hbm.at[pl.ds(c * _N + 15 * _ZR, _WLAST)])


def _dot(a, b):
    return jnp.dot(a, b, preferred_element_type=_f32)


def _full(shape):
    return pl.BlockSpec(shape, lambda *a: tuple(0 for _ in shape))


def _rows(shape):
    return pl.BlockSpec(shape, lambda i: (i,) + tuple(0 for _ in shape[1:]))


def _encoder_call(nodes, We, be, Wproj):
    def body(n_ref, we_ref, be_ref, wp_ref, x_ref, t_ref):
        x = _dot(n_ref[...], we_ref[...]) + be_ref[...]
        x_ref[...] = x
        t_ref[...] = _dot(x, wp_ref[...])

    return pl.pallas_call(
        body,
        grid=(_N // _NBLK,),
        in_specs=[_rows((_NBLK, 128)), _full((128, _DH)), _full((1, _DH)),
                  _full((_DH, 2 * _DH))],
        out_specs=[_rows((_NBLK, _DH)), _rows((_NBLK, 2 * _DH))],
        out_shape=[jax.ShapeDtypeStruct((_N, _DH), _f32),
                   jax.ShapeDtypeStruct((_N, 2 * _DH), _f32)],
    )(nodes, We, be.reshape(1, _DH), Wproj)


def _edge_call(e, gs, gr, wb, ln, d_store):
    (W1e, b1), (W2, b2), (W3, b3) = wb

    def body(*refs):
        if ln is None:
            (e_ref, gs_ref, gr_ref, w1_ref, b1_ref, w2_ref, b2_ref,
             w3_ref, b3_ref, out_ref) = refs
            ein = e_ref[...]
        else:
            (e_ref, gs_ref, gr_ref, lns_ref, lnb_ref, w1_ref, b1_ref,
             w2_ref, b2_ref, w3_ref, b3_ref, out_ref) = refs
            ein = _ln(e_ref[:, :_DH], lns_ref[...], lnb_ref[...])
        g = gs_ref[:, :_DH] + gr_ref[:, _DH:]
        h = _dot(ein, w1_ref[...]) + g + b1_ref[...]
        h = jax.nn.gelu(h)
        h = jax.nn.gelu(_dot(h, w2_ref[...]) + b2_ref[...])
        h = _dot(h, w3_ref[...]) + b3_ref[...]
        out_ref[...] = jnp.concatenate([h, jnp.zeros_like(h)], axis=-1)

    d_in = W1e.shape[0]
    ins = [e, gs, gr]
    in_specs = [_rows((_EBLK, d_store)),
                _rows((_EBLK, 2 * _DH)), _rows((_EBLK, 2 * _DH))]
    if ln is not None:
        ins += [ln[0].reshape(1, _DH), ln[1].reshape(1, _DH)]
        in_specs += [_full((1, _DH))] * 2
    ins += [W1e, b1.reshape(1, _DH), W2, b2.reshape(1, _DH),
            W3, b3.reshape(1, _DH)]
    in_specs += [_full((d_in, _DH)), _full((1, _DH)), _full((_DH, _DH)),
                 _full((1, _DH)), _full((_DH, _DH)), _full((1, _DH))]
    return pl.pallas_call(
        body,
        grid=(_E // _EBLK,),
        in_specs=in_specs,
        out_specs=_rows((_EBLK, 2 * _DH)),
        out_shape=jax.ShapeDtypeStruct((_E, 2 * _DH), _f32),
    )(*ins)


def _node_call(x, parts, wb, lnn, proj):
    (W1, b1), (W2, b2), (W3, b3) = wb
    W1x, W1r = W1[:_DH], W1[_DH:]

    def body(*refs):
        if proj is None:
            (x_ref, p0_ref, p1_ref, w1x_ref, w1r_ref, b1_ref, w2_ref, b2_ref,
             w3_ref, b3_ref, lns_ref, lnb_ref, xo_ref) = refs
        else:
            (x_ref, p0_ref, p1_ref, w1x_ref, w1r_ref, b1_ref, w2_ref, b2_ref,
             w3_ref, b3_ref, lns_ref, lnb_ref, wp_ref,
             xo_ref, t_ref) = refs
        recv = p0_ref[:, :_DH] + p1_ref[:, :_DH]
        h = jax.nn.gelu(_dot(x_ref[...], w1x_ref[...])
                        + _dot(recv, w1r_ref[...]) + b1_ref[...])
        h = jax.nn.gelu(_dot(h, w2_ref[...]) + b2_ref[...])
        xn = _dot(h, w3_ref[...]) + b3_ref[...]
        xo = _ln(xn, lns_ref[...], lnb_ref[...])
        xo_ref[...] = xo
        if proj is not None:
            t_ref[...] = _dot(xo, wp_ref[...])

    nb = _N // _NBLK
    ins = [x, parts, parts, W1x, W1r, b1.reshape(1, _DH),
           W2, b2.reshape(1, _DH), W3, b3.reshape(1, _DH),
           lnn[0].reshape(1, _DH), lnn[1].reshape(1, _DH)]
    in_specs = ([_rows((_NBLK, _DH)),
                 pl.BlockSpec((_NBLK, 2 * _DH), lambda i: (i, 0)),
                 pl.BlockSpec((_NBLK, 2 * _DH), lambda i: (i + nb, 0))]
                + [_full((_DH, _DH)), _full((_DH, _DH)), _full((1, _DH)),
                   _full((_DH, _DH)), _full((1, _DH)), _full((_DH, _DH)),
                   _full((1, _DH)), _full((1, _DH)), _full((1, _DH))])
    o = jax.ShapeDtypeStruct((_N, _DH), _f32)
    if proj is None:
        out_specs, out_shape = _rows((_NBLK, _DH)), o
    else:
        ins += [proj]
        in_specs += [_full((_DH, 2 * _DH))]
        out_specs = [_rows((_NBLK, _DH)), _rows((_NBLK, 2 * _DH))]
        out_shape = [o, jax.ShapeDtypeStruct((_N, 2 * _DH), _f32)]
    return pl.pallas_call(
        body,
        grid=(_N // _NBLK,),
        in_specs=in_specs,
        out_specs=out_specs,
        out_shape=out_shape,
    )(*ins)


def _readout_call(x, ps):
    (W1, b1), (W2, b2), (W3, b3), (W4, b4) = ps

    def body(x_ref, w1_ref, b1_ref, w2_ref, b2_ref, w3_ref, b3_ref,
             w4t_ref, b4_ref, out_ref):
        agg = jnp.mean(x_ref[...], axis=0, keepdims=True)
        h = jax.nn.gelu(_dot(agg, w1_ref[...]) + b1_ref[...])
        h = jax.nn.gelu(_dot(h, w2_ref[...]) + b2_ref[...])
        h = jax.nn.gelu(_dot(h, w3_ref[...]) + b3_ref[...])
        y = jnp.sum(h * w4t_ref[...]) + b4_ref[0, 0]
        out_ref[...] = jnp.full((1, 128), y, _f32)

    out = pl.pallas_call(
        body,
        in_specs=[_full((_N, _DH)), _full((_DH, 4 * _DH)), _full((1, 4 * _DH)),
                  _full((4 * _DH, 2 * _DH)), _full((1, 2 * _DH)),
                  _full((2 * _DH, 2 * _DH)), _full((1, 2 * _DH)),
                  _full((1, 2 * _DH)), _full((1, 1))],
        out_specs=_full((1, 128)),
        out_shape=jax.ShapeDtypeStruct((1, 128), _f32),
    )(x, W1, b1.reshape(1, -1), W2, b2.reshape(1, -1), W3, b3.reshape(1, -1),
      W4.reshape(1, -1), b4.reshape(1, 1))
    return out[0, :1]


def kernel(nodes, edge_attr, params, edge_index):
    senders = edge_index[0].reshape(_NW, _RPW)
    receivers = edge_index[1].reshape(_NW, _RPW)
    recv_sc = edge_index[1].reshape(_NW, _NCH, _CH)
    steps = params['steps']

    def _proj_w(W1, d_e):
        return jnp.concatenate([W1[d_e:d_e + _DH], W1[d_e + _DH:]], axis=1)

    w1_0 = steps[0]['edge'][0][0]
    x, tab = _encoder_call(nodes, params['enc'][0], params['enc'][1],
                           _proj_w(w1_0, 16))

    e = edge_attr
    for s in range(len(steps)):
        stp = steps[s]
        d_e = 16 if s == 0 else _DH
        d_store = 16 if s == 0 else 2 * _DH
        (W1, b1), l2, l3 = stp['edge']
        wb = ((W1[:d_e], b1), l2, l3)
        ln = None if s == 0 else (steps[s - 1]['ln_e'][0], steps[s - 1]['ln_e'][1])

        gs, gr = _sc_gather(tab, senders, receivers)
        e = _edge_call(e, gs, gr, wb, ln, d_store)
        parts = _sc_scatter(e, recv_sc)
        if s + 1 < len(steps):
            proj = _proj_w(steps[s + 1]['edge'][0][0], _DH)
            x, tab = _node_call(x, parts, stp['node'], stp['ln_n'], proj)
        else:
            x = _node_call(x, parts, stp['node'], stp['ln_n'], None)

    return _readout_call(x, params['readout'])


# speedup vs baseline: 3.2392x; 1.0353x over previous
import functools

import jax
import jax.numpy as jnp
from jax import lax
from jax.experimental import pallas as pl
from jax.experimental.pallas import tpu as pltpu
from jax.experimental.pallas import tpu_sc as plsc

_N = 10000
_E = 320000
_DH = 64
_NW = 32
_RPW = _E // _NW
_CH = 80
_NCH = _RPW // _CH
_NB = 5
_SNB = 2
_NPAD = 10240
_ZR = _NPAD // 16
_WLAST = _N - 15 * _ZR

_NBLK = 1000
_EBLK = 4000

_f32 = jnp.float32


def _ln(x, scale, bias):
    mu = jnp.mean(x, axis=-1, keepdims=True)
    var = jnp.mean((x - mu) * (x - mu), axis=-1, keepdims=True)
    return (x - mu) / jnp.sqrt(var + 1e-6) * scale + bias


_sc_mesh = plsc.VectorSubcoreMesh(core_axis_name="c", subcore_axis_name="s")


@functools.partial(
    pl.kernel,
    mesh=_sc_mesh,
    out_type=[
        jax.ShapeDtypeStruct((_E, 2 * _DH), _f32),
        jax.ShapeDtypeStruct((_E, 2 * _DH), _f32),
    ],
    scratch_types=(
        [pltpu.VMEM((_RPW,), jnp.int32)] * 2
        + [pltpu.VMEM((_CH, 2 * _DH), _f32)] * (2 * _NB)
        + [pltpu.SemaphoreType.DMA] * (4 * _NB)
    ),
)
def _sc_gather(tab_hbm, sidx_hbm, ridx_hbm, gs_hbm, gr_hbm,
               sidx_v, ridx_v, *rest):
    bufs = rest[:_NB]
    bufr = rest[_NB:2 * _NB]
    gsem_s = rest[2 * _NB:3 * _NB]
    gsem_r = rest[3 * _NB:4 * _NB]
    wsem_s = rest[4 * _NB:5 * _NB]
    wsem_r = rest[5 * _NB:6 * _NB]

    c = lax.axis_index("c")
    s = lax.axis_index("s")
    w = c * 16 + s
    base = w * _RPW

    pltpu.sync_copy(sidx_hbm.at[w], sidx_v)
    pltpu.sync_copy(ridx_hbm.at[w], ridx_v)

    def fire_gather(j, b):
        ds = pl.ds(j * _CH, _CH)
        pltpu.async_copy(tab_hbm.at[sidx_v.at[ds]], bufs[b], gsem_s[b])
        pltpu.async_copy(tab_hbm.at[ridx_v.at[ds]], bufr[b], gsem_r[b])

    def wait_gather(j, b):
        ds = pl.ds(j * _CH, _CH)
        pltpu.make_async_copy(tab_hbm.at[sidx_v.at[ds]], bufs[b], gsem_s[b]).wait()
        pltpu.make_async_copy(tab_hbm.at[ridx_v.at[ds]], bufr[b], gsem_r[b]).wait()

    def fire_write(j, b):
        dst = pl.ds(base + j * _CH, _CH)
        pltpu.async_copy(bufs[b], gs_hbm.at[dst], wsem_s[b])
        pltpu.async_copy(bufr[b], gr_hbm.at[dst], wsem_r[b])

    def wait_write(j, b):
        dst = pl.ds(base + j * _CH, _CH)
        pltpu.make_async_copy(bufs[b], gs_hbm.at[dst], wsem_s[b]).wait()
        pltpu.make_async_copy(bufr[b], gr_hbm.at[dst], wsem_r[b]).wait()

    for b in range(_NB):
        fire_gather(b, b)

    def outer(g, carry):
        for b in range(_NB):
            j = g * _NB + b
            wait_gather(j, b)
            fire_write(j, b)
        for b in range(_NB):
            j = g * _NB + b
            wait_write(j, b)

            @pl.when(j + _NB < _NCH)
            def _():
                fire_gather(j + _NB, b)
        return carry

    lax.fori_loop(0, _NCH // _NB, outer, 0)


@functools.partial(
    pl.kernel,
    mesh=_sc_mesh,
    out_type=jax.ShapeDtypeStruct((2 * _N, 2 * _DH), _f32),
    scratch_types=(
        [pltpu.VMEM((_NCH, _CH), jnp.int32),
         pltpu.VMEM((_CH, 2 * _DH), _f32)]
        + [pltpu.VMEM((_CH, 2 * _DH), _f32)] * _SNB
        + [pltpu.VMEM_SHARED((_NPAD, 2 * _DH), _f32)]
        + [pltpu.SemaphoreType.DMA] * _SNB
    ),
)
def _sc_scatter(e_hbm, ridx_hbm, out_hbm, ridx_v, zbuf, *rest):
    bufs = rest[:_SNB]
    shared = rest[_SNB]
    lsem = rest[_SNB + 1:]

    c = lax.axis_index("c")
    s = lax.axis_index("s")
    w = c * 16 + s
    base = w * _RPW

    pltpu.sync_copy(ridx_hbm.at[w], ridx_v)

    def zrow(i, carry):
        for q in range(8):
            zbuf[i, pl.ds(q * 16, 16)] = jnp.zeros((16,), _f32)
        return carry

    lax.fori_loop(0, _CH, zrow, 0)
    for q in range(_ZR // _CH):
        pltpu.sync_copy(zbuf, shared.at[pl.ds(s * _ZR + q * _CH, _CH)])
    plsc.subcore_barrier()

    def fire_load(j, b):
        pltpu.async_copy(e_hbm.at[pl.ds(base + j * _CH, _CH)], bufs[b], lsem[b])

    def wait_load(j, b):
        pltpu.make_async_copy(
            e_hbm.at[pl.ds(base + j * _CH, _CH)], bufs[b], lsem[b]).wait()

    for b in range(_SNB):
        fire_load(b, b)

    def outer(g, carry):
        for b in range(_SNB):
            j = g * _SNB + b

            @pl.when(j < _NCH)
            def _():
                wait_load(j, b)
                pltpu.sync_copy(bufs[b], shared.at[ridx_v.at[j]], add=True)

            @pl.when(j + _SNB < _NCH)
            def _():
                fire_load(j + _SNB, b)
        return carry

    lax.fori_loop(0, (_NCH + _SNB - 1) // _SNB, outer, 0)

    plsc.subcore_barrier()

    @pl.when(s < 15)
    def _():
        pltpu.sync_copy(shared.at[pl.ds(s * _ZR, _ZR)],
                        out_hbm.at[pl.ds(c * _N + s * _ZR, _ZR)])

    @pl.when(s == 15)
    def _():
        pltpu.sync_copy(shared.at[pl.ds(15 * _ZR, _WLAST)],
                        out_hbm.at[pl.ds(c * _N + 15 * _ZR, _WLAST)])


def _dot(a, b):
    return jnp.dot(a, b, preferred_element_type=_f32)


def _full(shape):
    return pl.BlockSpec(shape, lambda *a: tuple(0 for _ in shape))


def _rows(shape):
    return pl.BlockSpec(shape, lambda i: (i,) + tuple(0 for _ in shape[1:]))


def _encoder_call(nodes, We, be, Wproj):
    def body(n_ref, we_ref, be_ref, wp_ref, x_ref, t_ref):
        x = _dot(n_ref[...], we_ref[...]) + be_ref[...]
        x_ref[...] = x
        t_ref[...] = _dot(x, wp_ref[...])

    return pl.pallas_call(
        body,
        grid=(_N // _NBLK,),
        in_specs=[_rows((_NBLK, 128)), _full((128, _DH)), _full((1, _DH)),
                  _full((_DH, 2 * _DH))],
        out_specs=[_rows((_NBLK, _DH)), _rows((_NBLK, 2 * _DH))],
        out_shape=[jax.ShapeDtypeStruct((_N, _DH), _f32),
                   jax.ShapeDtypeStruct((_N, 2 * _DH), _f32)],
    )(nodes, We, be.reshape(1, _DH), Wproj)


def _edge_call(e, gs, gr, wb, ln, d_store):
    (W1e, b1), (W2, b2), (W3, b3) = wb

    def body(*refs):
        if ln is None:
            (e_ref, gs_ref, gr_ref, w1_ref, b1_ref, w2_ref, b2_ref,
             w3_ref, b3_ref, out_ref) = refs
            ein = e_ref[...]
        else:
            (e_ref, gs_ref, gr_ref, lns_ref, lnb_ref, w1_ref, b1_ref,
             w2_ref, b2_ref, w3_ref, b3_ref, out_ref) = refs
            ein = _ln(e_ref[:, :_DH], lns_ref[...], lnb_ref[...])
        g = gs_ref[:, :_DH] + gr_ref[:, _DH:]
        h = _dot(ein, w1_ref[...]) + g + b1_ref[...]
        h = jax.nn.gelu(h)
        h = jax.nn.gelu(_dot(h, w2_ref[...]) + b2_ref[...])
        h = _dot(h, w3_ref[...]) + b3_ref[...]
        out_ref[...] = jnp.concatenate([h, jnp.zeros_like(h)], axis=-1)

    d_in = W1e.shape[0]
    ins = [e, gs, gr]
    in_specs = [_rows((_EBLK, d_store)),
                _rows((_EBLK, 2 * _DH)), _rows((_EBLK, 2 * _DH))]
    if ln is not None:
        ins += [ln[0].reshape(1, _DH), ln[1].reshape(1, _DH)]
        in_specs += [_full((1, _DH))] * 2
    ins += [W1e, b1.reshape(1, _DH), W2, b2.reshape(1, _DH),
            W3, b3.reshape(1, _DH)]
    in_specs += [_full((d_in, _DH)), _full((1, _DH)), _full((_DH, _DH)),
                 _full((1, _DH)), _full((_DH, _DH)), _full((1, _DH))]
    return pl.pallas_call(
        body,
        grid=(_E // _EBLK,),
        in_specs=in_specs,
        out_specs=_rows((_EBLK, 2 * _DH)),
        out_shape=jax.ShapeDtypeStruct((_E, 2 * _DH), _f32),
    )(*ins)


def _node_call(x, parts, wb, lnn, proj):
    (W1, b1), (W2, b2), (W3, b3) = wb
    W1x, W1r = W1[:_DH], W1[_DH:]

    def body(*refs):
        if proj is None:
            (x_ref, p0_ref, p1_ref, w1x_ref, w1r_ref, b1_ref, w2_ref, b2_ref,
             w3_ref, b3_ref, lns_ref, lnb_ref, xo_ref) = refs
        else:
            (x_ref, p0_ref, p1_ref, w1x_ref, w1r_ref, b1_ref, w2_ref, b2_ref,
             w3_ref, b3_ref, lns_ref, lnb_ref, wp_ref,
             xo_ref, t_ref) = refs
        recv = p0_ref[:, :_DH] + p1_ref[:, :_DH]
        h = jax.nn.gelu(_dot(x_ref[...], w1x_ref[...])
                        + _dot(recv, w1r_ref[...]) + b1_ref[...])
        h = jax.nn.gelu(_dot(h, w2_ref[...]) + b2_ref[...])
        xn = _dot(h, w3_ref[...]) + b3_ref[...]
        xo = _ln(xn, lns_ref[...], lnb_ref[...])
        xo_ref[...] = xo
        if proj is not None:
            t_ref[...] = _dot(xo, wp_ref[...])

    nb = _N // _NBLK
    ins = [x, parts, parts, W1x, W1r, b1.reshape(1, _DH),
           W2, b2.reshape(1, _DH), W3, b3.reshape(1, _DH),
           lnn[0].reshape(1, _DH), lnn[1].reshape(1, _DH)]
    in_specs = ([_rows((_NBLK, _DH)),
                 pl.BlockSpec((_NBLK, 2 * _DH), lambda i: (i, 0)),
                 pl.BlockSpec((_NBLK, 2 * _DH), lambda i: (i + nb, 0))]
                + [_full((_DH, _DH)), _full((_DH, _DH)), _full((1, _DH)),
                   _full((_DH, _DH)), _full((1, _DH)), _full((_DH, _DH)),
                   _full((1, _DH)), _full((1, _DH)), _full((1, _DH))])
    o = jax.ShapeDtypeStruct((_N, _DH), _f32)
    if proj is None:
        out_specs, out_shape = _rows((_NBLK, _DH)), o
    else:
        ins += [proj]
        in_specs += [_full((_DH, 2 * _DH))]
        out_specs = [_rows((_NBLK, _DH)), _rows((_NBLK, 2 * _DH))]
        out_shape = [o, jax.ShapeDtypeStruct((_N, 2 * _DH), _f32)]
    return pl.pallas_call(
        body,
        grid=(_N // _NBLK,),
        in_specs=in_specs,
        out_specs=out_specs,
        out_shape=out_shape,
    )(*ins)


def _readout_call(x, ps):
    (W1, b1), (W2, b2), (W3, b3), (W4, b4) = ps

    def body(x_ref, w1_ref, b1_ref, w2_ref, b2_ref, w3_ref, b3_ref,
             w4t_ref, b4_ref, out_ref):
        agg = jnp.mean(x_ref[...], axis=0, keepdims=True)
        h = jax.nn.gelu(_dot(agg, w1_ref[...]) + b1_ref[...])
        h = jax.nn.gelu(_dot(h, w2_ref[...]) + b2_ref[...])
        h = jax.nn.gelu(_dot(h, w3_ref[...]) + b3_ref[...])
        y = jnp.sum(h * w4t_ref[...]) + b4_ref[0, 0]
        out_ref[...] = jnp.full((1, 128), y, _f32)

    out = pl.pallas_call(
        body,
        in_specs=[_full((_N, _DH)), _full((_DH, 4 * _DH)), _full((1, 4 * _DH)),
                  _full((4 * _DH, 2 * _DH)), _full((1, 2 * _DH)),
                  _full((2 * _DH, 2 * _DH)), _full((1, 2 * _DH)),
                  _full((1, 2 * _DH)), _full((1, 1))],
        out_specs=_full((1, 128)),
        out_shape=jax.ShapeDtypeStruct((1, 128), _f32),
    )(x, W1, b1.reshape(1, -1), W2, b2.reshape(1, -1), W3, b3.reshape(1, -1),
      W4.reshape(1, -1), b4.reshape(1, 1))
    return out[0, :1]


def kernel(nodes, edge_attr, params, edge_index):
    senders = edge_index[0].reshape(_NW, _RPW)
    receivers = edge_index[1].reshape(_NW, _RPW)
    recv_sc = edge_index[1].reshape(_NW, _NCH, _CH)
    steps = params['steps']

    def _proj_w(W1, d_e):
        return jnp.concatenate([W1[d_e:d_e + _DH], W1[d_e + _DH:]], axis=1)

    w1_0 = steps[0]['edge'][0][0]
    x, tab = _encoder_call(nodes, params['enc'][0], params['enc'][1],
                           _proj_w(w1_0, 16))

    e = edge_attr
    for s in range(len(steps)):
        stp = steps[s]
        d_e = 16 if s == 0 else _DH
        d_store = 16 if s == 0 else 2 * _DH
        (W1, b1), l2, l3 = stp['edge']
        wb = ((W1[:d_e], b1), l2, l3)
        ln = None if s == 0 else (steps[s - 1]['ln_e'][0], steps[s - 1]['ln_e'][1])

        gs, gr = _sc_gather(tab, senders, receivers)
        e = _edge_call(e, gs, gr, wb, ln, d_store)
        parts = _sc_scatter(e, recv_sc)
        if s + 1 < len(steps):
            proj = _proj_w(steps[s + 1]['edge'][0][0], _DH)
            x, tab = _node_call(x, parts, stp['node'], stp['ln_n'], proj)
        else:
            x = _node_call(x, parts, stp['node'], stp['ln_n'], None)

    return _readout_call(x, params['readout'])


# speedup vs baseline: 3.8223x; 1.1800x over previous
import functools

import jax
import jax.numpy as jnp
from jax import lax
from jax.experimental import pallas as pl
from jax.experimental.pallas import tpu as pltpu
from jax.experimental.pallas import tpu_sc as plsc

_N = 10000
_E = 320000
_DH = 64
_NW = 32
_RPW = _E // _NW
_CH = 80
_NCH = _RPW // _CH
_GCH = 40
_GNCH = _RPW // _GCH
_NB = 2
_SNB = 2
_NPAD = 10240
_ZR = _NPAD // 16
_WLAST = _N - 15 * _ZR

_NBLK = 1000
_EBLK = 4000

_f32 = jnp.float32


def _ln(x, scale, bias):
    mu = jnp.mean(x, axis=-1, keepdims=True)
    var = jnp.mean((x - mu) * (x - mu), axis=-1, keepdims=True)
    return (x - mu) / jnp.sqrt(var + 1e-6) * scale + bias


_sc_mesh = plsc.VectorSubcoreMesh(core_axis_name="c", subcore_axis_name="s")


@functools.partial(
    pl.kernel,
    mesh=_sc_mesh,
    out_type=[
        jax.ShapeDtypeStruct((_E, 2 * _DH), _f32),
        jax.ShapeDtypeStruct((_E, 2 * _DH), _f32),
    ],
    scratch_types=(
        [pltpu.VMEM((_RPW,), jnp.int32)] * 2
        + [pltpu.VMEM((_GCH, 2 * _DH), _f32)] * (2 * _NB)
        + [pltpu.VMEM_SHARED((_NPAD, 2 * _DH), _f32)]
        + [pltpu.SemaphoreType.DMA] * (4 * _NB)
    ),
)
def _sc_gather(tab_hbm, sidx_hbm, ridx_hbm, gs_hbm, gr_hbm,
               sidx_v, ridx_v, *rest):
    bufs = rest[:_NB]
    bufr = rest[_NB:2 * _NB]
    tab_sp = rest[2 * _NB]
    gsem_s = rest[2 * _NB + 1:3 * _NB + 1]
    gsem_r = rest[3 * _NB + 1:4 * _NB + 1]
    wsem_s = rest[4 * _NB + 1:5 * _NB + 1]
    wsem_r = rest[5 * _NB + 1:6 * _NB + 1]

    c = lax.axis_index("c")
    s = lax.axis_index("s")
    w = c * 16 + s
    base = w * _RPW

    @pl.when(s < 15)
    def _():
        pltpu.sync_copy(tab_hbm.at[pl.ds(s * _ZR, _ZR)],
                        tab_sp.at[pl.ds(s * _ZR, _ZR)])

    @pl.when(s == 15)
    def _():
        pltpu.sync_copy(tab_hbm.at[pl.ds(15 * _ZR, _WLAST)],
                        tab_sp.at[pl.ds(15 * _ZR, _WLAST)])

    pltpu.sync_copy(sidx_hbm.at[w], sidx_v)
    pltpu.sync_copy(ridx_hbm.at[w], ridx_v)
    plsc.subcore_barrier()

    def fire_gather(j, b):
        ds = pl.ds(j * _GCH, _GCH)
        pltpu.async_copy(tab_sp.at[sidx_v.at[ds]], bufs[b], gsem_s[b])
        pltpu.async_copy(tab_sp.at[ridx_v.at[ds]], bufr[b], gsem_r[b])

    def wait_gather(j, b):
        ds = pl.ds(j * _GCH, _GCH)
        pltpu.make_async_copy(tab_sp.at[sidx_v.at[ds]], bufs[b], gsem_s[b]).wait()
        pltpu.make_async_copy(tab_sp.at[ridx_v.at[ds]], bufr[b], gsem_r[b]).wait()

    def fire_write(j, b):
        dst = pl.ds(base + j * _GCH, _GCH)
        pltpu.async_copy(bufs[b], gs_hbm.at[dst], wsem_s[b])
        pltpu.async_copy(bufr[b], gr_hbm.at[dst], wsem_r[b])

    def wait_write(j, b):
        dst = pl.ds(base + j * _GCH, _GCH)
        pltpu.make_async_copy(bufs[b], gs_hbm.at[dst], wsem_s[b]).wait()
        pltpu.make_async_copy(bufr[b], gr_hbm.at[dst], wsem_r[b]).wait()

    for b in range(_NB):
        fire_gather(b, b)

    def outer(g, carry):
        for b in range(_NB):
            j = g * _NB + b
            wait_gather(j, b)
            fire_write(j, b)
        for b in range(_NB):
            j = g * _NB + b
            wait_write(j, b)

            @pl.when(j + _NB < _GNCH)
            def _():
                fire_gather(j + _NB, b)
        return carry

    lax.fori_loop(0, _GNCH // _NB, outer, 0)


@functools.partial(
    pl.kernel,
    mesh=_sc_mesh,
    out_type=jax.ShapeDtypeStruct((2 * _N, 2 * _DH), _f32),
    scratch_types=(
        [pltpu.VMEM((_NCH, _CH), jnp.int32),
         pltpu.VMEM((_CH, 2 * _DH), _f32)]
        + [pltpu.VMEM((_CH, 2 * _DH), _f32)] * _SNB
        + [pltpu.VMEM_SHARED((_NPAD, 2 * _DH), _f32)]
        + [pltpu.SemaphoreType.DMA] * _SNB
    ),
)
def _sc_scatter(e_hbm, ridx_hbm, out_hbm, ridx_v, zbuf, *rest):
    bufs = rest[:_SNB]
    shared = rest[_SNB]
    lsem = rest[_SNB + 1:]

    c = lax.axis_index("c")
    s = lax.axis_index("s")
    w = c * 16 + s
    base = w * _RPW

    pltpu.sync_copy(ridx_hbm.at[w], ridx_v)

    def zrow(i, carry):
        for q in range(8):
            zbuf[i, pl.ds(q * 16, 16)] = jnp.zeros((16,), _f32)
        return carry

    lax.fori_loop(0, _CH, zrow, 0)
    for q in range(_ZR // _CH):
        pltpu.sync_copy(zbuf, shared.at[pl.ds(s * _ZR + q * _CH, _CH)])
    plsc.subcore_barrier()

    def fire_load(j, b):
        pltpu.async_copy(e_hbm.at[pl.ds(base + j * _CH, _CH)], bufs[b], lsem[b])

    def wait_load(j, b):
        pltpu.make_async_copy(
            e_hbm.at[pl.ds(base + j * _CH, _CH)], bufs[b], lsem[b]).wait()

    for b in range(_SNB):
        fire_load(b, b)

    def outer(g, carry):
        for b in range(_SNB):
            j = g * _SNB + b

            @pl.when(j < _NCH)
            def _():
                wait_load(j, b)
                pltpu.sync_copy(bufs[b], shared.at[ridx_v.at[j]], add=True)

            @pl.when(j + _SNB < _NCH)
            def _():
                fire_load(j + _SNB, b)
        return carry

    lax.fori_loop(0, (_NCH + _SNB - 1) // _SNB, outer, 0)

    plsc.subcore_barrier()

    @pl.when(s < 15)
    def _():
        pltpu.sync_copy(shared.at[pl.ds(s * _ZR, _ZR)],
                        out_hbm.at[pl.ds(c * _N + s * _ZR, _ZR)])

    @pl.when(s == 15)
    def _():
        pltpu.sync_copy(shared.at[pl.ds(15 * _ZR, _WLAST)],
                        out_hbm.at[pl.ds(c * _N + 15 * _ZR, _WLAST)])


def _dot(a, b):
    return jnp.dot(a, b, preferred_element_type=_f32)


def _full(shape):
    return pl.BlockSpec(shape, lambda *a: tuple(0 for _ in shape))


def _rows(shape):
    return pl.BlockSpec(shape, lambda i: (i,) + tuple(0 for _ in shape[1:]))


def _encoder_call(nodes, We, be, Wproj):
    def body(n_ref, we_ref, be_ref, wp_ref, x_ref, t_ref):
        x = _dot(n_ref[...], we_ref[...]) + be_ref[...]
        x_ref[...] = x
        t_ref[...] = _dot(x, wp_ref[...])

    return pl.pallas_call(
        body,
        grid=(_N // _NBLK,),
        in_specs=[_rows((_NBLK, 128)), _full((128, _DH)), _full((1, _DH)),
                  _full((_DH, 2 * _DH))],
        out_specs=[_rows((_NBLK, _DH)), _rows((_NBLK, 2 * _DH))],
        out_shape=[jax.ShapeDtypeStruct((_N, _DH), _f32),
                   jax.ShapeDtypeStruct((_N, 2 * _DH), _f32)],
    )(nodes, We, be.reshape(1, _DH), Wproj)


def _edge_call(e, gs, gr, wb, ln, d_store):
    (W1e, b1), (W2, b2), (W3, b3) = wb

    def body(*refs):
        if ln is None:
            (e_ref, gs_ref, gr_ref, w1_ref, b1_ref, w2_ref, b2_ref,
             w3_ref, b3_ref, out_ref) = refs
            ein = e_ref[...]
        else:
            (e_ref, gs_ref, gr_ref, lns_ref, lnb_ref, w1_ref, b1_ref,
             w2_ref, b2_ref, w3_ref, b3_ref, out_ref) = refs
            ein = _ln(e_ref[:, :_DH], lns_ref[...], lnb_ref[...])
        g = gs_ref[:, :_DH] + gr_ref[:, _DH:]
        h = _dot(ein, w1_ref[...]) + g + b1_ref[...]
        h = jax.nn.gelu(h)
        h = jax.nn.gelu(_dot(h, w2_ref[...]) + b2_ref[...])
        h = _dot(h, w3_ref[...]) + b3_ref[...]
        out_ref[...] = jnp.concatenate([h, jnp.zeros_like(h)], axis=-1)

    d_in = W1e.shape[0]
    ins = [e, gs, gr]
    in_specs = [_rows((_EBLK, d_store)),
                _rows((_EBLK, 2 * _DH)), _rows((_EBLK, 2 * _DH))]
    if ln is not None:
        ins += [ln[0].reshape(1, _DH), ln[1].reshape(1, _DH)]
        in_specs += [_full((1, _DH))] * 2
    ins += [W1e, b1.reshape(1, _DH), W2, b2.reshape(1, _DH),
            W3, b3.reshape(1, _DH)]
    in_specs += [_full((d_in, _DH)), _full((1, _DH)), _full((_DH, _DH)),
                 _full((1, _DH)), _full((_DH, _DH)), _full((1, _DH))]
    return pl.pallas_call(
        body,
        grid=(_E // _EBLK,),
        in_specs=in_specs,
        out_specs=_rows((_EBLK, 2 * _DH)),
        out_shape=jax.ShapeDtypeStruct((_E, 2 * _DH), _f32),
    )(*ins)


def _node_call(x, parts, wb, lnn, proj):
    (W1, b1), (W2, b2), (W3, b3) = wb
    W1x, W1r = W1[:_DH], W1[_DH:]

    def body(*refs):
        if proj is None:
            (x_ref, p0_ref, p1_ref, w1x_ref, w1r_ref, b1_ref, w2_ref, b2_ref,
             w3_ref, b3_ref, lns_ref, lnb_ref, xo_ref) = refs
        else:
            (x_ref, p0_ref, p1_ref, w1x_ref, w1r_ref, b1_ref, w2_ref, b2_ref,
             w3_ref, b3_ref, lns_ref, lnb_ref, wp_ref,
             xo_ref, t_ref) = refs
        recv = p0_ref[:, :_DH] + p1_ref[:, :_DH]
        h = jax.nn.gelu(_dot(x_ref[...], w1x_ref[...])
                        + _dot(recv, w1r_ref[...]) + b1_ref[...])
        h = jax.nn.gelu(_dot(h, w2_ref[...]) + b2_ref[...])
        xn = _dot(h, w3_ref[...]) + b3_ref[...]
        xo = _ln(xn, lns_ref[...], lnb_ref[...])
        xo_ref[...] = xo
        if proj is not None:
            t_ref[...] = _dot(xo, wp_ref[...])

    nb = _N // _NBLK
    ins = [x, parts, parts, W1x, W1r, b1.reshape(1, _DH),
           W2, b2.reshape(1, _DH), W3, b3.reshape(1, _DH),
           lnn[0].reshape(1, _DH), lnn[1].reshape(1, _DH)]
    in_specs = ([_rows((_NBLK, _DH)),
                 pl.BlockSpec((_NBLK, 2 * _DH), lambda i: (i, 0)),
                 pl.BlockSpec((_NBLK, 2 * _DH), lambda i: (i + nb, 0))]
                + [_full((_DH, _DH)), _full((_DH, _DH)), _full((1, _DH)),
                   _full((_DH, _DH)), _full((1, _DH)), _full((_DH, _DH)),
                   _full((1, _DH)), _full((1, _DH)), _full((1, _DH))])
    o = jax.ShapeDtypeStruct((_N, _DH), _f32)
    if proj is None:
        out_specs, out_shape = _rows((_NBLK, _DH)), o
    else:
        ins += [proj]
        in_specs += [_full((_DH, 2 * _DH))]
        out_specs = [_rows((_NBLK, _DH)), _rows((_NBLK, 2 * _DH))]
        out_shape = [o, jax.ShapeDtypeStruct((_N, 2 * _DH), _f32)]
    return pl.pallas_call(
        body,
        grid=(_N // _NBLK,),
        in_specs=in_specs,
        out_specs=out_specs,
        out_shape=out_shape,
    )(*ins)


def _readout_call(x, ps):
    (W1, b1), (W2, b2), (W3, b3), (W4, b4) = ps

    def body(x_ref, w1_ref, b1_ref, w2_ref, b2_ref, w3_ref, b3_ref,
             w4t_ref, b4_ref, out_ref):
        agg = jnp.mean(x_ref[...], axis=0, keepdims=True)
        h = jax.nn.gelu(_dot(agg, w1_ref[...]) + b1_ref[...])
        h = jax.nn.gelu(_dot(h, w2_ref[...]) + b2_ref[...])
        h = jax.nn.gelu(_dot(h, w3_ref[...]) + b3_ref[...])
        y = jnp.sum(h * w4t_ref[...]) + b4_ref[0, 0]
        out_ref[...] = jnp.full((1, 128), y, _f32)

    out = pl.pallas_call(
        body,
        in_specs=[_full((_N, _DH)), _full((_DH, 4 * _DH)), _full((1, 4 * _DH)),
                  _full((4 * _DH, 2 * _DH)), _full((1, 2 * _DH)),
                  _full((2 * _DH, 2 * _DH)), _full((1, 2 * _DH)),
                  _full((1, 2 * _DH)), _full((1, 1))],
        out_specs=_full((1, 128)),
        out_shape=jax.ShapeDtypeStruct((1, 128), _f32),
    )(x, W1, b1.reshape(1, -1), W2, b2.reshape(1, -1), W3, b3.reshape(1, -1),
      W4.reshape(1, -1), b4.reshape(1, 1))
    return out[0, :1]


def kernel(nodes, edge_attr, params, edge_index):
    senders = edge_index[0].reshape(_NW, _RPW)
    receivers = edge_index[1].reshape(_NW, _RPW)
    recv_sc = edge_index[1].reshape(_NW, _NCH, _CH)
    steps = params['steps']

    def _proj_w(W1, d_e):
        return jnp.concatenate([W1[d_e:d_e + _DH], W1[d_e + _DH:]], axis=1)

    w1_0 = steps[0]['edge'][0][0]
    x, tab = _encoder_call(nodes, params['enc'][0], params['enc'][1],
                           _proj_w(w1_0, 16))

    e = edge_attr
    for s in range(len(steps)):
        stp = steps[s]
        d_e = 16 if s == 0 else _DH
        d_store = 16 if s == 0 else 2 * _DH
        (W1, b1), l2, l3 = stp['edge']
        wb = ((W1[:d_e], b1), l2, l3)
        ln = None if s == 0 else (steps[s - 1]['ln_e'][0], steps[s - 1]['ln_e'][1])

        gs, gr = _sc_gather(tab, senders, receivers)
        e = _edge_call(e, gs, gr, wb, ln, d_store)
        parts = _sc_scatter(e, recv_sc)
        if s + 1 < len(steps):
            proj = _proj_w(steps[s + 1]['edge'][0][0], _DH)
            x, tab = _node_call(x, parts, stp['node'], stp['ln_n'], proj)
        else:
            x = _node_call(x, parts, stp['node'], stp['ln_n'], None)

    return _readout_call(x, params['readout'])
